# baseline (speedup 1.0000x reference)
import functools

import jax
import jax.numpy as jnp
from jax import lax
from jax.experimental import pallas as pl
from jax.experimental.pallas import tpu as pltpu

_VMEM_LIMIT = 48 * 1024 * 1024


def _conv_stats_kernel(x_ref, w_ref, conv_ref, sum_ref, sq_ref, *,
                       th, out_w, cin, kh_size, kw_size, t_tiles, n_samp):
    mt = th * out_w
    for i in range(n_samp):
        s = None
        for t in range(t_tiles):
            acc = None
            for kw in range(kw_size):
                pieces = []
                for kh in range(kh_size):
                    tap = x_ref[i, pl.ds(t * th + kh, th), pl.ds(kw, out_w), :]
                    pieces.append(tap.reshape(mt, cin))
                rhs = jnp.concatenate(pieces, axis=1)
                d = jnp.dot(rhs, w_ref[kw],
                            preferred_element_type=jnp.float32)
                acc = d if acc is None else acc + d

            conv_ref[i, :, t * mt:(t + 1) * mt] = acc.astype(conv_ref.dtype).T
            if s is None:
                s = jnp.sum(acc, axis=0, keepdims=True)
                q = jnp.sum(acc * acc, axis=0, keepdims=True)
            else:
                s = s + jnp.sum(acc, axis=0, keepdims=True)
                q = q + jnp.sum(acc * acc, axis=0, keepdims=True)

        sum_ref[i] = s
        sq_ref[i] = q


def _bn_apply_kernel(c_ref, scale_ref, shift_ref, o_ref):
    for i in range(c_ref.shape[0]):
        y = c_ref[i].astype(jnp.float32)
        o_ref[i] = (y * scale_ref[...] + shift_ref[...]).astype(o_ref.dtype)


def kernel(x_nchw, weight_oihw, gamma, beta, *, eps=1e-5):
    N, Cin, H, W = x_nchw.shape
    Cout, _, KH, KW = weight_oihw.shape
    oh, ow = H, W
    m_total = oh * ow
    pad_h = KH - 1
    pad_w = KW - 1

    x = jnp.transpose(x_nchw, (0, 2, 3, 1)).astype(jnp.bfloat16)
    x = jnp.pad(x, ((0, 0),
                    (pad_h // 2, pad_h - pad_h // 2),
                    (pad_w // 2, pad_w - pad_w // 2),
                    (0, 0)))
    hp, wp = x.shape[1], x.shape[2]

    w3 = jnp.transpose(weight_oihw, (3, 2, 1, 0)).reshape(KW, KH * Cin, Cout)
    w3 = w3.astype(jnp.bfloat16)

    T = 4
    while oh % T:
        T -= 1
    th = oh // T

    S = 8
    while N % S:
        S -= 1

    cparams = pltpu.CompilerParams(
        dimension_semantics=("parallel",),
        vmem_limit_bytes=_VMEM_LIMIT)

    conv_kernel = functools.partial(
        _conv_stats_kernel, th=th, out_w=ow, cin=Cin,
        kh_size=KH, kw_size=KW, t_tiles=T, n_samp=S)

    conv_flat, psum, psq = pl.pallas_call(
        conv_kernel,
        grid=(N // S,),
        in_specs=[
            pl.BlockSpec((S, hp, wp, Cin), lambda n: (n, 0, 0, 0)),
            pl.BlockSpec((KW, KH * Cin, Cout), lambda n: (0, 0, 0)),
        ],
        out_specs=(
            pl.BlockSpec((S, Cout, m_total), lambda n: (n, 0, 0)),
            pl.BlockSpec((S, 1, Cout), lambda n: (n, 0, 0)),
            pl.BlockSpec((S, 1, Cout), lambda n: (n, 0, 0)),
        ),
        out_shape=(
            jax.ShapeDtypeStruct((N, Cout, m_total), jnp.bfloat16),
            jax.ShapeDtypeStruct((N, 1, Cout), jnp.float32),
            jax.ShapeDtypeStruct((N, 1, Cout), jnp.float32),
        ),
        compiler_params=cparams,
    )(x, w3)

    count = float(N * m_total)
    mean = jnp.sum(psum, axis=0) / count
    var = jnp.maximum(jnp.sum(psq, axis=0) / count - mean * mean, 0.0)
    inv = lax.rsqrt(var + eps)
    gamma32 = gamma.astype(jnp.float32).reshape(1, Cout)
    beta32 = beta.astype(jnp.float32).reshape(1, Cout)
    scale = (gamma32 * inv).reshape(Cout, 1)
    shift = (beta32 - mean * gamma32 * inv).reshape(Cout, 1)

    out_flat = pl.pallas_call(
        _bn_apply_kernel,
        grid=(N // S,),
        in_specs=[
            pl.BlockSpec((S, Cout, m_total), lambda n: (n, 0, 0)),
            pl.BlockSpec((Cout, 1), lambda n: (0, 0)),
            pl.BlockSpec((Cout, 1), lambda n: (0, 0)),
        ],
        out_specs=pl.BlockSpec((S, Cout, m_total), lambda n: (n, 0, 0)),
        out_shape=jax.ShapeDtypeStruct((N, Cout, m_total), x_nchw.dtype),
        compiler_params=pltpu.CompilerParams(
            dimension_semantics=("parallel",),
            vmem_limit_bytes=_VMEM_LIMIT),
    )(conv_flat, scale, shift)

    return out_flat.reshape(N, Cout, oh, ow)


# speedup vs baseline: 1.2704x; 1.2704x over previous
import functools

import jax
import jax.numpy as jnp
from jax import lax
from jax.experimental import pallas as pl
from jax.experimental.pallas import tpu as pltpu

_VMEM_LIMIT = 56 * 1024 * 1024


def _conv_stats_kernel(x_ref, w_ref, conv_ref, sum_ref, sq_ref, *,
                       th, out_w, cin, kh_size, kw_size, t_tiles, n_samp):
    mt = th * out_w
    for i in range(n_samp):
        s = None
        for t in range(t_tiles):
            acc = None
            for kw in range(kw_size):
                pieces = []
                for kh in range(kh_size):
                    tap = x_ref[i, pl.ds(t * th + kh, th), pl.ds(kw, out_w), :]
                    pieces.append(tap.reshape(mt, cin).astype(jnp.bfloat16))
                rhs = jnp.concatenate(pieces, axis=1)
                d = jnp.dot(rhs, w_ref[kw],
                            preferred_element_type=jnp.float32)
                acc = d if acc is None else acc + d

            conv_ref[i, :, t * mt:(t + 1) * mt] = acc.astype(conv_ref.dtype).T
            if s is None:
                s = jnp.sum(acc, axis=0, keepdims=True)
                q = jnp.sum(acc * acc, axis=0, keepdims=True)
            else:
                s = s + jnp.sum(acc, axis=0, keepdims=True)
                q = q + jnp.sum(acc * acc, axis=0, keepdims=True)

        sum_ref[i] = s
        sq_ref[i] = q


def _bn_apply_kernel(c_ref, scale_ref, shift_ref, o_ref):
    for i in range(c_ref.shape[0]):
        y = c_ref[i].astype(jnp.float32)
        o_ref[i] = (y * scale_ref[...] + shift_ref[...]).astype(o_ref.dtype)


def kernel(x_nchw, weight_oihw, gamma, beta, *, eps=1e-5):
    N, Cin, H, W = x_nchw.shape
    Cout, _, KH, KW = weight_oihw.shape
    oh, ow = H, W
    m_total = oh * ow
    pad_h = KH - 1
    pad_w = KW - 1

    x = jnp.transpose(x_nchw, (0, 2, 3, 1))
    x = jnp.pad(x, ((0, 0),
                    (pad_h // 2, pad_h - pad_h // 2),
                    (pad_w // 2, pad_w - pad_w // 2),
                    (0, 0)))
    hp, wp = x.shape[1], x.shape[2]

    w3 = jnp.transpose(weight_oihw, (3, 2, 1, 0)).reshape(KW, KH * Cin, Cout)
    w3 = w3.astype(jnp.bfloat16)

    T = 2
    while oh % T:
        T -= 1
    th = oh // T

    S = 8
    while N % S:
        S -= 1

    cparams = pltpu.CompilerParams(
        dimension_semantics=("parallel",),
        vmem_limit_bytes=_VMEM_LIMIT)

    conv_kernel = functools.partial(
        _conv_stats_kernel, th=th, out_w=ow, cin=Cin,
        kh_size=KH, kw_size=KW, t_tiles=T, n_samp=S)

    conv_flat, psum, psq = pl.pallas_call(
        conv_kernel,
        grid=(N // S,),
        in_specs=[
            pl.BlockSpec((S, hp, wp, Cin), lambda n: (n, 0, 0, 0)),
            pl.BlockSpec((KW, KH * Cin, Cout), lambda n: (0, 0, 0)),
        ],
        out_specs=(
            pl.BlockSpec((S, Cout, m_total), lambda n: (n, 0, 0)),
            pl.BlockSpec((S, 1, Cout), lambda n: (n, 0, 0)),
            pl.BlockSpec((S, 1, Cout), lambda n: (n, 0, 0)),
        ),
        out_shape=(
            jax.ShapeDtypeStruct((N, Cout, m_total), jnp.bfloat16),
            jax.ShapeDtypeStruct((N, 1, Cout), jnp.float32),
            jax.ShapeDtypeStruct((N, 1, Cout), jnp.float32),
        ),
        compiler_params=cparams,
    )(x, w3)

    count = float(N * m_total)
    mean = jnp.sum(psum, axis=0) / count
    var = jnp.maximum(jnp.sum(psq, axis=0) / count - mean * mean, 0.0)
    inv = lax.rsqrt(var + eps)
    gamma32 = gamma.astype(jnp.float32).reshape(1, Cout)
    beta32 = beta.astype(jnp.float32).reshape(1, Cout)
    scale = (gamma32 * inv).reshape(Cout, 1)
    shift = (beta32 - mean * gamma32 * inv).reshape(Cout, 1)

    out_flat = pl.pallas_call(
        _bn_apply_kernel,
        grid=(N // S,),
        in_specs=[
            pl.BlockSpec((S, Cout, m_total), lambda n: (n, 0, 0)),
            pl.BlockSpec((Cout, 1), lambda n: (0, 0)),
            pl.BlockSpec((Cout, 1), lambda n: (0, 0)),
        ],
        out_specs=pl.BlockSpec((S, Cout, m_total), lambda n: (n, 0, 0)),
        out_shape=jax.ShapeDtypeStruct((N, Cout, m_total), x_nchw.dtype),
        compiler_params=pltpu.CompilerParams(
            dimension_semantics=("parallel",),
            vmem_limit_bytes=_VMEM_LIMIT),
    )(conv_flat, scale, shift)

    return out_flat.reshape(N, Cout, oh, ow)


# speedup vs baseline: 1.3024x; 1.0252x over previous
import functools

import jax
import jax.numpy as jnp
from jax import lax
from jax.experimental import pallas as pl
from jax.experimental.pallas import tpu as pltpu

_VMEM_LIMIT = 48 * 1024 * 1024


def _conv_stats_kernel(x_ref, w_ref, conv_ref, sum_ref, sq_ref, *,
                       th, out_w, cin, kh_size, kw_size, t_tiles, n_samp):
    mt = th * out_w
    for i in range(n_samp):
        s = None
        for t in range(t_tiles):
            acc = None
            for kw in range(kw_size):
                pieces = []
                for kh in range(kh_size):
                    tap = x_ref[i, pl.ds(t * th + kh, th), pl.ds(kw, out_w), :]
                    pieces.append(tap.reshape(mt, cin).astype(jnp.bfloat16))
                rhs = jnp.concatenate(pieces, axis=1)
                d = jnp.dot(rhs, w_ref[kw],
                            preferred_element_type=jnp.float32)
                acc = d if acc is None else acc + d

            conv_ref[i, :, t * mt:(t + 1) * mt] = acc.astype(conv_ref.dtype).T
            if s is None:
                s = jnp.sum(acc, axis=0, keepdims=True)
                q = jnp.sum(acc * acc, axis=0, keepdims=True)
            else:
                s = s + jnp.sum(acc, axis=0, keepdims=True)
                q = q + jnp.sum(acc * acc, axis=0, keepdims=True)

        sum_ref[i] = s
        sq_ref[i] = q


def _bn_apply_kernel(c_ref, scale_ref, shift_ref, o_ref):
    for i in range(c_ref.shape[0]):
        y = c_ref[i].astype(jnp.float32)
        o_ref[i] = (y * scale_ref[...] + shift_ref[...]).astype(o_ref.dtype)


def kernel(x_nchw, weight_oihw, gamma, beta, *, eps=1e-5):
    N, Cin, H, W = x_nchw.shape
    Cout, _, KH, KW = weight_oihw.shape
    oh, ow = H, W
    m_total = oh * ow
    pad_h = KH - 1
    pad_w = KW - 1

    x = jnp.transpose(x_nchw, (0, 2, 3, 1))
    x = jnp.pad(x, ((0, 0),
                    (pad_h // 2, pad_h - pad_h // 2),
                    (pad_w // 2, pad_w - pad_w // 2),
                    (0, 0)))
    hp, wp = x.shape[1], x.shape[2]

    w3 = jnp.transpose(weight_oihw, (3, 2, 1, 0)).reshape(KW, KH * Cin, Cout)
    w3 = w3.astype(jnp.bfloat16)

    T = 4
    while oh % T:
        T -= 1
    th = oh // T

    S = 8
    while N % S:
        S -= 1

    cparams = pltpu.CompilerParams(
        dimension_semantics=("parallel",),
        vmem_limit_bytes=_VMEM_LIMIT)

    conv_kernel = functools.partial(
        _conv_stats_kernel, th=th, out_w=ow, cin=Cin,
        kh_size=KH, kw_size=KW, t_tiles=T, n_samp=S)

    conv_flat, psum, psq = pl.pallas_call(
        conv_kernel,
        grid=(N // S,),
        in_specs=[
            pl.BlockSpec((S, hp, wp, Cin), lambda n: (n, 0, 0, 0)),
            pl.BlockSpec((KW, KH * Cin, Cout), lambda n: (0, 0, 0)),
        ],
        out_specs=(
            pl.BlockSpec((S, Cout, m_total), lambda n: (n, 0, 0)),
            pl.BlockSpec((S, 1, Cout), lambda n: (n, 0, 0)),
            pl.BlockSpec((S, 1, Cout), lambda n: (n, 0, 0)),
        ),
        out_shape=(
            jax.ShapeDtypeStruct((N, Cout, m_total), jnp.bfloat16),
            jax.ShapeDtypeStruct((N, 1, Cout), jnp.float32),
            jax.ShapeDtypeStruct((N, 1, Cout), jnp.float32),
        ),
        compiler_params=cparams,
    )(x, w3)

    count = float(N * m_total)
    mean = jnp.sum(psum, axis=0) / count
    var = jnp.maximum(jnp.sum(psq, axis=0) / count - mean * mean, 0.0)
    inv = lax.rsqrt(var + eps)
    gamma32 = gamma.astype(jnp.float32).reshape(1, Cout)
    beta32 = beta.astype(jnp.float32).reshape(1, Cout)
    scale = (gamma32 * inv).reshape(Cout, 1)
    shift = (beta32 - mean * gamma32 * inv).reshape(Cout, 1)

    out_flat = pl.pallas_call(
        _bn_apply_kernel,
        grid=(N // S,),
        in_specs=[
            pl.BlockSpec((S, Cout, m_total), lambda n: (n, 0, 0)),
            pl.BlockSpec((Cout, 1), lambda n: (0, 0)),
            pl.BlockSpec((Cout, 1), lambda n: (0, 0)),
        ],
        out_specs=pl.BlockSpec((S, Cout, m_total), lambda n: (n, 0, 0)),
        out_shape=jax.ShapeDtypeStruct((N, Cout, m_total), x_nchw.dtype),
        compiler_params=pltpu.CompilerParams(
            dimension_semantics=("parallel",),
            vmem_limit_bytes=_VMEM_LIMIT),
    )(conv_flat, scale, shift)

    return out_flat.reshape(N, Cout, oh, ow)
